# baseline (speedup 1.0000x reference)
import functools

import jax
import jax.numpy as jnp
from jax import lax
from jax.experimental import pallas as pl
from jax.experimental.pallas import tpu as pltpu
from jax.experimental.pallas import tpu_sc as plsc

VOCAB = 1000
VPAD = 1024
BATCH = 4096
BLOCK = 8
N = BATCH * BLOCK
NC = 2
NS = 16
NW = NC * NS
ROWS_PER_W = N // NW
CHUNK = 32
NCHUNK = ROWS_PER_W // CHUNK
NBUF = 3
BANDS_PER_CHUNK = CHUNK // 8

_mesh = plsc.VectorSubcoreMesh(core_axis_name="c", subcore_axis_name="s")


@functools.partial(
    pl.kernel,
    mesh=_mesh,
    out_type=(),
    scratch_types=[
        pltpu.VMEM((ROWS_PER_W,), jnp.int32),
        pltpu.VMEM((CHUNK, 8, 128), jnp.float32),
        pltpu.VMEM((CHUNK, 8, 128), jnp.float32),
        pltpu.VMEM((CHUNK, 8, 128), jnp.float32),
        pltpu.SemaphoreType.DMA,
        pltpu.SemaphoreType.DMA,
        pltpu.SemaphoreType.DMA,
        pltpu.SemaphoreType.DMA,
        pltpu.SemaphoreType.DMA,
        pltpu.SemaphoreType.DMA,
    ],
    compiler_params=pltpu.CompilerParams(use_tc_tiling_on_sc=False),
)
def _gather_kernel(
    w_hbm, idx_hbm, out_hbm, idx_v, b0, b1, b2, gs0, gs1, gs2, ss0, ss1, ss2
):
    wid = lax.axis_index("s") * NC + lax.axis_index("c")
    base_band = wid * (ROWS_PER_W // 8)
    pltpu.sync_copy(idx_hbm.at[pl.ds(wid * ROWS_PER_W, ROWS_PER_W)], idx_v)
    bufs = (b0, b1, b2)
    gsems = (gs0, gs1, gs2)
    ssems = (ss0, ss1, ss2)

    def gather(j):
        slot = j % NBUF
        return pltpu.async_copy(
            w_hbm.at[idx_v.at[pl.ds(j * CHUNK, CHUNK)]], bufs[slot], gsems[slot]
        )

    def scatter(j):
        slot = j % NBUF
        return [
            pltpu.async_copy(
                bufs[slot].at[pl.ds(8 * bb, 8)],
                out_hbm.at[base_band + j * BANDS_PER_CHUNK + bb],
                ssems[slot],
            )
            for bb in range(BANDS_PER_CHUNK)
        ]

    g = [None] * NCHUNK
    s = [None] * NCHUNK
    waited = [False] * NCHUNK
    g[0] = gather(0)
    g[1] = gather(1)
    for j in range(NCHUNK):
        if j + 2 < NCHUNK:
            if j >= 1:
                for h in s[j - 1]:
                    h.wait()
                waited[j - 1] = True
            g[j + 2] = gather(j + 2)
        g[j].wait()
        s[j] = scatter(j)
    for j in range(NCHUNK):
        if not waited[j]:
            for h in s[j]:
                h.wait()


def kernel(idx, W):
    w4 = jnp.pad(W, ((0, 0), (0, VPAD - VOCAB))).reshape(VOCAB, 8, 128)
    flat = idx.reshape(N).astype(jnp.int32)
    out_ref = jax.new_ref(jnp.zeros((BATCH, BLOCK, 8, 128), jnp.float32))
    _gather_kernel(w4, flat, out_ref)
    out = out_ref[...]
    return out.reshape(BATCH, BLOCK, VPAD)[:, :, :VOCAB]
